# Initial kernel scaffold; baseline (speedup 1.0000x reference)
import functools

import jax
import jax.numpy as jnp
from jax import lax
from jax.experimental import pallas as pl
from jax.experimental.pallas import tpu as pltpu
from jax.experimental.pallas import tpu_sc as plsc

N = 10000
E = 320000
H = 128
MAXDEG = 64
ZR = 0.8

_B = 128
_NC = 2
_NS = 16
_NW = _NC * _NS
_CW = -(-E // (_NW * _B))
_EPAD = _NW * _CW * _B
_RPT = N // _NS
_DT = 5
_DC = N // _DT


def _make_spmm(do_deg):
  mesh = plsc.VectorSubcoreMesh(core_axis_name="c", subcore_axis_name="s")
  out_type = [jax.ShapeDtypeStruct((_NC, N, H), jnp.float32)]
  scratch = [
      pltpu.VMEM((_CW, _B), jnp.int32),
      pltpu.VMEM((_CW, _B), jnp.int32),
      pltpu.VMEM((_CW, _B), jnp.float32),
      pltpu.VMEM((_B, H), jnp.float32),
      pltpu.VMEM_SHARED((N, H), jnp.float32),
      pltpu.SemaphoreType.DMA,
  ]
  if do_deg:
    out_type.append(jax.ShapeDtypeStruct((_NC, N), jnp.float32))
    scratch += [
        pltpu.VMEM((_DC,), jnp.float32),
        pltpu.VMEM_SHARED((N,), jnp.float32),
    ]

  @functools.partial(
      pl.kernel,
      out_type=tuple(out_type),
      mesh=mesh,
      scratch_types=tuple(scratch),
  )
  def spmm(xm_hbm, col3, row3, ew3, *refs):
    if do_deg:
      (part, degp, col_v, row_v, ew_v, rows_v, agg_sp, gsem,
       zdeg_v, deg_sp) = refs
    else:
      part, col_v, row_v, ew_v, rows_v, agg_sp, gsem = refs
    cid = lax.axis_index("c")
    sid = lax.axis_index("s")
    wid = cid * _NS + sid

    pltpu.sync_copy(col3.at[wid], col_v)
    pltpu.sync_copy(row3.at[wid], row_v)
    pltpu.sync_copy(ew3.at[wid], ew_v)

    zero16 = jnp.zeros((16,), jnp.float32)

    def zrow(r, carry):
      for t in range(H // 16):
        rows_v[r, pl.ds(t * 16, 16)] = zero16
      return carry

    lax.fori_loop(0, _B, zrow, 0)
    base = sid * _RPT
    for k in range(_RPT // _B):
      pltpu.sync_copy(rows_v, agg_sp.at[pl.ds(base + k * _B, _B)])
    rem = _RPT % _B
    if rem:
      pltpu.sync_copy(rows_v.at[pl.ds(0, rem)],
                      agg_sp.at[pl.ds(base + (_RPT // _B) * _B, rem)])

    if do_deg:
      def zd(i, carry):
        zdeg_v[pl.ds(i * 16, 16)] = zero16
        return carry

      lax.fori_loop(0, _DC // 16, zd, 0)

      @pl.when(sid < _DT)
      def _():
        pltpu.sync_copy(zdeg_v, deg_sp.at[pl.ds(sid * _DC, _DC)])

    plsc.subcore_barrier()

    def chunk(j, carry):
      pltpu.async_copy(xm_hbm.at[col_v.at[j]], rows_v, gsem).wait()

      def scale(e, c2):
        w = ew_v[j, e]
        wv = jnp.full((16,), w, jnp.float32)
        for t in range(H // 16):
          sl = pl.ds(t * 16, 16)
          rows_v[e, sl] = rows_v[e, sl] * wv
        return c2

      lax.fori_loop(0, _B, scale, 0)
      pltpu.sync_copy(rows_v, agg_sp.at[row_v.at[j]], add=True)
      if do_deg:
        pltpu.sync_copy(ew_v.at[j], deg_sp.at[row_v.at[j]], add=True)
      return carry

    lax.fori_loop(0, _CW, chunk, 0)
    plsc.subcore_barrier()

    pltpu.sync_copy(agg_sp.at[pl.ds(base, _RPT)],
                    part.at[cid, pl.ds(base, _RPT)])
    if do_deg:
      @pl.when(sid < _DT)
      def _():
        pltpu.sync_copy(deg_sp.at[pl.ds(sid * _DC, _DC)],
                        degp.at[cid, pl.ds(sid * _DC, _DC)])

  return spmm


_spmm_deg = _make_spmm(True)
_spmm = _make_spmm(False)


def _gnorm(v, w, b, ms):
  mean = jnp.mean(v, axis=0, keepdims=True)
  out = v - mean * ms
  var = jnp.mean(out * out, axis=0, keepdims=True)
  return w * out * lax.rsqrt(var + 1e-6) + b


def _mmT(a, w):
  return lax.dot_general(a, w, (((1,), (1,)), ((), ())),
                         preferred_element_type=jnp.float32)


def _k0_body(x_ref, z_ref, emb_ref, gw_ref, gb_ref, gms_ref,
             wt1_ref, bt1_ref, wt0_ref, bt0_ref, h_out, xm_out):
  xi = x_ref[...]
  iota = lax.broadcasted_iota(jnp.int32, (N, H), 1)
  oh = (xi == iota).astype(jnp.float32)
  h = jnp.dot(oh, emb_ref[...], preferred_element_type=jnp.float32)
  h = _gnorm(h, gw_ref[...], gb_ref[...], gms_ref[...])
  x1 = jax.nn.relu(_mmT(h, wt1_ref[...]) + bt1_ref[...])
  x0 = jax.nn.relu(_mmT(h, wt0_ref[...]) + bt0_ref[...])
  m = jnp.where(z_ref[...] > 0.5, ZR, 1.0 - ZR)
  h_out[...] = h
  xm_out[...] = x0 + m * (x1 - x0)


def _post_common(p_ref, dpair_ref, z_ref, h_ref, cgn, wc1_ref, bc1_ref,
                 wc0_ref, bc0_ref):
  deg = dpair_ref[:, 0:1] + dpair_ref[:, 1:2]
  deg = jnp.where(deg < 0.5, deg + 1.0, deg)
  agg = (p_ref[0] + p_ref[1]) / deg
  agg = _gnorm(agg, *cgn)
  h = h_ref[...]
  wc1 = wc1_ref[...]
  wc0 = wc0_ref[...]
  x1 = _mmT(agg, wc1[:, :H]) + _mmT(h, wc1[:, H:]) + bc1_ref[...]
  x0 = _mmT(agg, wc0[:, :H]) + _mmT(h, wc0[:, H:]) + bc0_ref[...]
  m = jnp.where(z_ref[...] > 0.5, ZR, 1.0 - ZR)
  return x0 + m * (x1 - x0), m


def _k2_body(p_ref, dpair_ref, z_ref, h_ref,
             cgw, cgb, cgms, wc1_ref, bc1_ref, wc0_ref, bc0_ref,
             gw, gb, gms, wt1_ref, bt1_ref, wt0_ref, bt0_ref,
             h_out, xm_out):
  hm, m = _post_common(p_ref, dpair_ref, z_ref, h_ref,
                       (cgw[...], cgb[...], cgms[...]),
                       wc1_ref, bc1_ref, wc0_ref, bc0_ref)
  h1 = jax.nn.relu(_gnorm(hm, gw[...], gb[...], gms[...]))
  y1 = jax.nn.relu(_mmT(h1, wt1_ref[...]) + bt1_ref[...])
  y0 = jax.nn.relu(_mmT(h1, wt0_ref[...]) + bt0_ref[...])
  h_out[...] = h1
  xm_out[...] = y0 + m * (y1 - y0)


def _k4_body(p_ref, dpair_ref, z_ref, h_ref,
             cgw, cgb, cgms, wc1_ref, bc1_ref, wc0_ref, bc0_ref,
             gw, gb, gms, out_ref):
  hm, _ = _post_common(p_ref, dpair_ref, z_ref, h_ref,
                       (cgw[...], cgb[...], cgms[...]),
                       wc1_ref, bc1_ref, wc0_ref, bc0_ref)
  out_ref[...] = _gnorm(hm, gw[...], gb[...], gms[...])


_NH = jax.ShapeDtypeStruct((N, H), jnp.float32)

_k0 = pl.pallas_call(_k0_body, out_shape=(_NH, _NH))
_k2 = pl.pallas_call(_k2_body, out_shape=(_NH, _NH))
_k4 = pl.pallas_call(_k4_body, out_shape=_NH)


@jax.jit
def kernel(x, edge_index, edge_weight, z, params):
  row = edge_index[0].astype(jnp.int32)
  col = edge_index[1].astype(jnp.int32)
  ew = edge_weight.astype(jnp.float32)
  pad = _EPAD - E
  col3 = jnp.pad(col, (0, pad)).reshape(_NW, _CW, _B)
  row3 = jnp.pad(row, (0, pad)).reshape(_NW, _CW, _B)
  ew3 = jnp.pad(ew, (0, pad)).reshape(_NW, _CW, _B)
  x2 = x.astype(jnp.int32).reshape(N, 1)
  z2 = z.astype(jnp.float32).reshape(N, 1)
  emb_pad = jnp.zeros((H, H), jnp.float32).at[:MAXDEG + 1].set(params["emb"])

  def v2(t):
    return tuple(a.reshape(1, H) for a in t)

  egw, egb, egms = v2(params["emb_gn"])
  h, xm0 = _k0(x2, z2, emb_pad, egw, egb, egms,
               params["t1_0"][0], params["t1_0"][1].reshape(1, H),
               params["t0_0"][0], params["t0_0"][1].reshape(1, H))

  p0, degp = _spmm_deg(xm0, col3, row3, ew3)
  dpair = degp.T.reshape(N, _NC)

  cg0 = v2(params["cgn_0"])
  g0 = v2(params["gn_0"])
  h1, xm1 = _k2(p0, dpair, z2, h,
                cg0[0], cg0[1], cg0[2],
                params["c1_0"][0], params["c1_0"][1].reshape(1, H),
                params["c0_0"][0], params["c0_0"][1].reshape(1, H),
                g0[0], g0[1], g0[2],
                params["t1_1"][0], params["t1_1"][1].reshape(1, H),
                params["t0_1"][0], params["t0_1"][1].reshape(1, H))

  p1 = _spmm(xm1, col3, row3, ew3)

  cg1 = v2(params["cgn_1"])
  g1 = v2(params["gn_1"])
  out = _k4(p1, dpair, z2, h1,
            cg1[0], cg1[1], cg1[2],
            params["c1_1"][0], params["c1_1"][1].reshape(1, H),
            params["c0_1"][0], params["c0_1"][1].reshape(1, H),
            g1[0], g1[1], g1[2])
  return out


# speedup vs baseline: 6.1377x; 6.1377x over previous
import functools

import jax
import jax.numpy as jnp
from jax import lax
from jax.experimental import pallas as pl
from jax.experimental.pallas import tpu as pltpu
from jax.experimental.pallas import tpu_sc as plsc

N = 10000
E = 320000
H = 128
MAXDEG = 64
ZR = 0.8

_B = 128
_NC = 2
_NS = 16
_NW = _NC * _NS
_CW = -(-E // (_NW * _B))
_EPAD = _NW * _CW * _B
_DT = 5
_DC = 2048
_DPAD = _DT * _DC


def _make_spmm(do_deg):
  mesh = plsc.VectorSubcoreMesh(core_axis_name="c", subcore_axis_name="s")
  out_type = [jax.ShapeDtypeStruct((_NC, N, H), jnp.float32)]
  scratch = [
      pltpu.VMEM((_CW, _B), jnp.int32),
      pltpu.VMEM((_CW, _B), jnp.int32),
      pltpu.VMEM((_CW, _B), jnp.float32),
      pltpu.VMEM((_B, H), jnp.float32),
      pltpu.VMEM_SHARED((N, H), jnp.float32),
      pltpu.SemaphoreType.DMA,
  ]
  if do_deg:
    out_type.append(jax.ShapeDtypeStruct((_DPAD,), jnp.float32))
    out_type.append(jax.ShapeDtypeStruct((_DPAD,), jnp.float32))
    scratch += [
        pltpu.VMEM((_DC,), jnp.float32),
        pltpu.VMEM_SHARED((_DPAD,), jnp.float32),
    ]

  @functools.partial(
      pl.kernel,
      out_type=tuple(out_type),
      mesh=mesh,
      scratch_types=tuple(scratch),
  )
  def spmm(xm_hbm, col3, row3, ew3, *refs):
    if do_deg:
      (part, deg0, deg1, col_v, row_v, ew_v, rows_v, agg_sp, gsem,
       zdeg_v, deg_sp) = refs
    else:
      part, col_v, row_v, ew_v, rows_v, agg_sp, gsem = refs
    cid = lax.axis_index("c")
    sid = lax.axis_index("s")
    wid = cid * _NS + sid

    pltpu.sync_copy(col3.at[wid], col_v)
    pltpu.sync_copy(row3.at[wid], row_v)
    pltpu.sync_copy(ew3.at[wid], ew_v)

    zero16 = jnp.zeros((16,), jnp.float32)

    def zrow(r, carry):
      for t in range(H // 16):
        rows_v[r, pl.ds(t * 16, 16)] = zero16
      return carry

    lax.fori_loop(0, _B, zrow, 0)
    base = pl.multiple_of(sid * 624, 8)

    def _zero_slice(start, nrows):
      for k in range(nrows // _B):
        pltpu.sync_copy(rows_v, agg_sp.at[pl.ds(start + k * _B, _B)])
      rem = nrows % _B
      if rem:
        pltpu.sync_copy(rows_v.at[pl.ds(0, rem)],
                        agg_sp.at[pl.ds(start + (nrows // _B) * _B, rem)])

    @pl.when(sid < _NS - 1)
    def _():
      _zero_slice(base, 624)

    @pl.when(sid == _NS - 1)
    def _():
      _zero_slice(base, 640)

    if do_deg:
      def zd(i, carry):
        zdeg_v[pl.ds(i * 16, 16)] = zero16
        return carry

      lax.fori_loop(0, _DC // 16, zd, 0)

      @pl.when(sid < _DT)
      def _():
        pltpu.sync_copy(zdeg_v, deg_sp.at[pl.ds(sid * _DC, _DC)])

    plsc.subcore_barrier()

    def chunk(j, carry):
      pltpu.async_copy(xm_hbm.at[col_v.at[j]], rows_v, gsem).wait()

      def scale(g, c2):
        wv16 = ew_v[j, pl.ds(g * 16, 16)]
        for i in range(16):
          wv = jnp.full((16,), wv16[i], jnp.float32)
          e = g * 16 + i
          for t in range(H // 16):
            sl = pl.ds(t * 16, 16)
            rows_v[e, sl] = rows_v[e, sl] * wv
        return c2

      lax.fori_loop(0, _B // 16, scale, 0)
      pltpu.sync_copy(rows_v, agg_sp.at[row_v.at[j]], add=True)
      if do_deg:
        pltpu.sync_copy(ew_v.at[j], deg_sp.at[row_v.at[j]], add=True)
      return carry

    lax.fori_loop(0, _CW, chunk, 0)
    plsc.subcore_barrier()

    @pl.when(sid < _NS - 1)
    def _():
      pltpu.sync_copy(agg_sp.at[pl.ds(base, 624)],
                      part.at[cid, pl.ds(base, 624)])

    @pl.when(sid == _NS - 1)
    def _():
      pltpu.sync_copy(agg_sp.at[pl.ds(base, 640)],
                      part.at[cid, pl.ds(base, 640)])
    if do_deg:
      @pl.when((sid < _DT) & (cid == 0))
      def _():
        pltpu.sync_copy(deg_sp.at[pl.ds(sid * _DC, _DC)],
                        deg0.at[pl.ds(sid * _DC, _DC)])

      @pl.when((sid < _DT) & (cid == 1))
      def _():
        pltpu.sync_copy(deg_sp.at[pl.ds(sid * _DC, _DC)],
                        deg1.at[pl.ds(sid * _DC, _DC)])

  return spmm


_spmm_deg = _make_spmm(True)
_spmm = _make_spmm(False)


def _gnorm(v, w, b, ms):
  mean = jnp.mean(v, axis=0, keepdims=True)
  out = v - mean * ms
  var = jnp.mean(out * out, axis=0, keepdims=True)
  return w * out * lax.rsqrt(var + 1e-6) + b


def _mmT(a, w):
  return lax.dot_general(a, w, (((1,), (1,)), ((), ())),
                         preferred_element_type=jnp.float32)


def _k0_body(x_ref, z_ref, emb_ref, gw_ref, gb_ref, gms_ref,
             wt1_ref, bt1_ref, wt0_ref, bt0_ref, h_out, xm_out):
  xi = x_ref[...]
  iota = lax.broadcasted_iota(jnp.int32, (N, H), 1)
  oh = (xi == iota).astype(jnp.float32)
  h = jnp.dot(oh, emb_ref[...], preferred_element_type=jnp.float32)
  h = _gnorm(h, gw_ref[...], gb_ref[...], gms_ref[...])
  x1 = jax.nn.relu(_mmT(h, wt1_ref[...]) + bt1_ref[...])
  x0 = jax.nn.relu(_mmT(h, wt0_ref[...]) + bt0_ref[...])
  m = jnp.where(z_ref[...] > 0.5, ZR, 1.0 - ZR)
  h_out[...] = h
  xm_out[...] = x0 + m * (x1 - x0)


def _post_common(p_ref, dpair_ref, z_ref, h_ref, cgn, wc1_ref, bc1_ref,
                 wc0_ref, bc0_ref):
  deg = dpair_ref[:, 0:1] + dpair_ref[:, 1:2]
  deg = jnp.where(deg < 0.5, deg + 1.0, deg)
  agg = (p_ref[0, :, :] + p_ref[1, :, :]) / deg
  agg = _gnorm(agg, *cgn)
  h = h_ref[...]
  wc1 = wc1_ref[...]
  wc0 = wc0_ref[...]
  x1 = _mmT(agg, wc1[:, :H]) + _mmT(h, wc1[:, H:]) + bc1_ref[...]
  x0 = _mmT(agg, wc0[:, :H]) + _mmT(h, wc0[:, H:]) + bc0_ref[...]
  m = jnp.where(z_ref[...] > 0.5, ZR, 1.0 - ZR)
  return x0 + m * (x1 - x0), m


def _k2_body(p_ref, dpair_ref, z_ref, h_ref,
             cgw, cgb, cgms, wc1_ref, bc1_ref, wc0_ref, bc0_ref,
             gw, gb, gms, wt1_ref, bt1_ref, wt0_ref, bt0_ref,
             h_out, xm_out):
  hm, m = _post_common(p_ref, dpair_ref, z_ref, h_ref,
                       (cgw[...], cgb[...], cgms[...]),
                       wc1_ref, bc1_ref, wc0_ref, bc0_ref)
  h1 = jax.nn.relu(_gnorm(hm, gw[...], gb[...], gms[...]))
  y1 = jax.nn.relu(_mmT(h1, wt1_ref[...]) + bt1_ref[...])
  y0 = jax.nn.relu(_mmT(h1, wt0_ref[...]) + bt0_ref[...])
  h_out[...] = h1
  xm_out[...] = y0 + m * (y1 - y0)


def _k4_body(p_ref, dpair_ref, z_ref, h_ref,
             cgw, cgb, cgms, wc1_ref, bc1_ref, wc0_ref, bc0_ref,
             gw, gb, gms, out_ref):
  hm, _ = _post_common(p_ref, dpair_ref, z_ref, h_ref,
                       (cgw[...], cgb[...], cgms[...]),
                       wc1_ref, bc1_ref, wc0_ref, bc0_ref)
  out_ref[...] = _gnorm(hm, gw[...], gb[...], gms[...])


_NH = jax.ShapeDtypeStruct((N, H), jnp.float32)

_k0 = pl.pallas_call(_k0_body, out_shape=(_NH, _NH))
_k2 = pl.pallas_call(_k2_body, out_shape=(_NH, _NH))
_k4 = pl.pallas_call(_k4_body, out_shape=_NH)


@jax.jit
def kernel(x, edge_index, edge_weight, z, params):
  row = edge_index[0].astype(jnp.int32)
  col = edge_index[1].astype(jnp.int32)
  ew = edge_weight.astype(jnp.float32)
  pad = _EPAD - E
  col3 = jnp.pad(col, (0, pad)).reshape(_NW, _CW, _B)
  row3 = jnp.pad(row, (0, pad)).reshape(_NW, _CW, _B)
  ew3 = jnp.pad(ew, (0, pad)).reshape(_NW, _CW, _B)
  x2 = x.astype(jnp.int32).reshape(N, 1)
  z2 = z.astype(jnp.float32).reshape(N, 1)
  emb_pad = jnp.zeros((H, H), jnp.float32).at[:MAXDEG + 1].set(params["emb"])

  def v2(t):
    return tuple(a.reshape(1, H) for a in t)

  egw, egb, egms = v2(params["emb_gn"])
  h, xm0 = _k0(x2, z2, emb_pad, egw, egb, egms,
               params["t1_0"][0], params["t1_0"][1].reshape(1, H),
               params["t0_0"][0], params["t0_0"][1].reshape(1, H))

  p0, d0, d1 = _spmm_deg(xm0, col3, row3, ew3)
  dpair = jnp.stack([d0[:N], d1[:N]], axis=1)

  cg0 = v2(params["cgn_0"])
  g0 = v2(params["gn_0"])
  h1, xm1 = _k2(p0, dpair, z2, h,
                cg0[0], cg0[1], cg0[2],
                params["c1_0"][0], params["c1_0"][1].reshape(1, H),
                params["c0_0"][0], params["c0_0"][1].reshape(1, H),
                g0[0], g0[1], g0[2],
                params["t1_1"][0], params["t1_1"][1].reshape(1, H),
                params["t0_1"][0], params["t0_1"][1].reshape(1, H))

  p1, = _spmm(xm1, col3, row3, ew3)

  cg1 = v2(params["cgn_1"])
  g1 = v2(params["gn_1"])
  out = _k4(p1, dpair, z2, h1,
            cg1[0], cg1[1], cg1[2],
            params["c1_1"][0], params["c1_1"][1].reshape(1, H),
            params["c0_1"][0], params["c0_1"][1].reshape(1, H),
            g1[0], g1[1], g1[2])
  return out
